# baseline (speedup 1.0000x reference)
import functools

import jax
import jax.numpy as jnp
import numpy as np
from jax import lax
from jax.experimental import pallas as pl
from jax.experimental.pallas import tpu as pltpu
from jax.experimental.pallas import tpu_sc as plsc

N = 10000
D = 256
DH = 128
E = 160000
EA = E + N
L = 16
NC = 2
NS = 16
EP = 180224
ER = EP // 128
RPT = ER // NS
CPT = 2 * RPT
BLK = 8
NP = 10240
NACC = NP // NS
NB = 10
BN = N // NB
PRESERVE = 0.1

_mesh = plsc.VectorSubcoreMesh(
    core_axis_name="c", subcore_axis_name="s", num_cores=NC, num_subcores=NS)
_sc_params = pltpu.CompilerParams(needs_layout_passes=False)


@functools.partial(
    pl.kernel,
    out_type=jax.ShapeDtypeStruct((NC, NP // 128, 128), jnp.float32),
    mesh=_mesh,
    scratch_types=[
        pltpu.VMEM((NP // 128, 128), jnp.float32),
        pltpu.VMEM((RPT, 128), jnp.int32),
        pltpu.VMEM((RPT, 128), jnp.float32),
        pltpu.VMEM_SHARED((NS, NP // 128, 128), jnp.float32),
        pltpu.VMEM((NS, 8, 128), jnp.float32),
        pltpu.VMEM((8, 128), jnp.float32),
    ],
    compiler_params=_sc_params,
)
def _deg_kernel(dst_hbm, w_hbm, deg_hbm, degloc, dstv, wv, degsh, redv, accv):
    c = lax.axis_index("c")
    s = lax.axis_index("s")
    zero = jnp.zeros((L,), jnp.float32)

    def zbody(i, _):
        for q in range(128 // L):
            degloc[i, pl.ds(q * L, L)] = zero
        return 0
    lax.fori_loop(0, NP // 128, zbody, 0)

    pltpu.sync_copy(dst_hbm.at[pl.ds(s * RPT, RPT)], dstv)
    pltpu.sync_copy(w_hbm.at[pl.ds(s * RPT, RPT)], wv)

    def ebody(j, _):
        for q in range(128 // L):
            idx = dstv[j, pl.ds(q * L, L)]
            wq = wv[j, pl.ds(q * L, L)]
            plsc.addupdate_scatter(
                degloc, [lax.shift_right_logical(idx, 7),
                         lax.bitwise_and(idx, 127)], wq)
        return 0
    lax.fori_loop(0, RPT, ebody, 0)

    pltpu.sync_copy(degloc, degsh.at[s])
    plsc.subcore_barrier()
    nrt = 8

    @pl.when(s < NP // 128 // nrt)
    def _reduce():
        pltpu.sync_copy(degsh.at[:, pl.ds(s * nrt, nrt)], redv)

        def rbody(v, _):
            for q in range(128 // L):
                acc = redv[0, v, pl.ds(q * L, L)]
                for r in range(1, NS):
                    acc = acc + redv[r, v, pl.ds(q * L, L)]
                accv[v, pl.ds(q * L, L)] = acc
            return 0
        lax.fori_loop(0, nrt, rbody, 0)
        pltpu.sync_copy(accv, deg_hbm.at[c, pl.ds(s * nrt, nrt)])


@functools.partial(
    pl.kernel,
    out_type=jax.ShapeDtypeStruct((NC, NP, DH), jnp.float32),
    mesh=_mesh,
    scratch_types=[
        pltpu.VMEM((RPT, 128), jnp.int32),
        pltpu.VMEM((BLK, 128), jnp.int32),
        pltpu.VMEM((BLK, 128), jnp.float32),
        pltpu.VMEM((128, DH), jnp.float32),
        pltpu.VMEM((128, DH), jnp.float32),
        pltpu.VMEM_SHARED((NP, DH), jnp.float32),
        pltpu.SemaphoreType.DMA,
        pltpu.SemaphoreType.DMA,
        pltpu.SemaphoreType.DMA,
        pltpu.SemaphoreType.DMA,
        pltpu.SemaphoreType.DMA,
        pltpu.SemaphoreType.DMA,
    ],
    compiler_params=_sc_params,
)
def _edge_kernel(h_hbm, src_hbm, dst_hbm, w_hbm, out_hbm,
                 srcv, dblk, wblk, big0, big1, acc,
                 gsem0, gsem1, gsem2, gsem3, ssem0, ssem1):
    c = lax.axis_index("c")
    s = lax.axis_index("s")
    zero = jnp.zeros((L,), jnp.float32)
    bigs = (big0, big1)
    gsem = (gsem0, gsem1, gsem2, gsem3)
    ssem = (ssem0, ssem1)

    def zb(i, _):
        for q in range(DH // L):
            big0[i, pl.ds(q * L, L)] = zero
        return 0
    lax.fori_loop(0, 128, zb, 0)
    for k in range(NACC // 128):
        pltpu.sync_copy(big0, acc.at[pl.ds(s * NACC + k * 128, 128)])

    pltpu.sync_copy(src_hbm.at[pl.ds(s * RPT, RPT)], srcv)

    coff = c * NP

    def ob(j, _):
        for q in range(128 // L):
            srcv[j, pl.ds(q * L, L)] = srcv[j, pl.ds(q * L, L)] + coff
        return 0
    lax.fori_loop(0, RPT, ob, 0)

    def _fire(jn, k):
        half = k % 2
        pltpu.async_copy(
            h_hbm.at[srcv.at[lax.div(jn, 2), pl.ds(half * 64, 64)]],
            bigs[k // 2].at[pl.ds(half * 64, 64)], gsem[k])

    def _gwait(k):
        pltpu.make_async_copy(h_hbm.at[pl.ds(0, 64)],
                              bigs[k // 2].at[pl.ds(0, 64)], gsem[k]).wait()

    def _swait(bi):
        pltpu.make_async_copy(h_hbm.at[pl.ds(0, 128)],
                              bigs[bi], ssem[bi]).wait()

    for k in range(2):
        _fire(k, k)

    plsc.subcore_barrier()

    def group(grp, _):
        godd = lax.rem(grp, 2) == 1
        for p in range(8):
            j = 8 * grp + p
            rowb = lax.rem(lax.div(j, 2), BLK)
            if p == 0:
                @pl.when(jnp.logical_not(godd))
                def _():
                    blk = lax.div(j, 2 * BLK)
                    pltpu.sync_copy(
                        dst_hbm.at[pl.ds(s * RPT + blk * BLK, BLK)], dblk)
                    pltpu.sync_copy(
                        w_hbm.at[pl.ds(s * RPT + blk * BLK, BLK)], wblk)
            if p == 0:
                @pl.when(godd)
                def _():
                    _swait(1)
            elif p == 4:
                _swait(1)
            elif p in (2, 6):
                _swait(0)
            if p < 6:
                _fire(j + 2, (p + 2) % 4)
            else:
                @pl.when(grp < CPT // 8 - 1)
                def _():
                    _fire(j + 2, (p + 2) % 4)
            _gwait(p % 4)
            bb = bigs[(p % 4) // 2]
            base = (p % 2) * 64
            jj = jnp.broadcast_to(rowb, (L,))

            def scale(r4, _2):
                for u in range(4):
                    r = base + r4 * 4 + u
                    rr = jnp.broadcast_to(r, (L,))
                    wspl = plsc.load_gather(wblk, [jj, rr])
                    for q in range(DH // L):
                        bb[r, pl.ds(q * L, L)] = bb[r, pl.ds(q * L, L)] * wspl
                return 0
            lax.fori_loop(0, 16, scale, 0)
            if p % 2 == 1:
                bi = (p % 4) // 2
                if p < 7:
                    pltpu.async_copy(bb, acc.at[dblk.at[rowb]], ssem[bi],
                                     add=True)
                else:
                    @pl.when(godd)
                    def _():
                        pltpu.sync_copy(bb, acc.at[dblk.at[rowb]], add=True)

                    @pl.when(jnp.logical_not(godd))
                    def _():
                        pltpu.async_copy(bb, acc.at[dblk.at[rowb]], ssem[bi],
                                         add=True)
        return 0
    lax.fori_loop(0, CPT // 8, group, 0)

    plsc.subcore_barrier()
    pltpu.sync_copy(acc.at[pl.ds(s * NACC, NACC)],
                    out_hbm.at[c, pl.ds(s * NACC, NACC)])


def _mm_body(x_ref, w_ref, deg_ref, out_ref):
    dinv = lax.rsqrt(deg_ref[...])
    h = jnp.dot(x_ref[...], w_ref[...], preferred_element_type=jnp.float32)
    out_ref[0] = h * dinv


BNM = 2000

_mm = pl.pallas_call(
    _mm_body,
    grid=(NC, N // BNM),
    in_specs=[
        pl.BlockSpec((BNM, D), lambda c, i: (i, 0)),
        pl.BlockSpec((D, DH), lambda c, i: (0, c)),
        pl.BlockSpec((BNM, 1), lambda c, i: (i, 0)),
    ],
    out_specs=pl.BlockSpec((1, BNM, DH), lambda c, i: (c, i, 0)),
    out_shape=jax.ShapeDtypeStruct((NC, NP, DH), jnp.float32),
)


def _fin_body(s_lo, s_hi, deg_ref, b_ref, xp_ref, out_ref):
    dinv = lax.rsqrt(deg_ref[...])
    o = dinv * jnp.concatenate([s_lo[0], s_hi[0]], axis=1) + b_ref[...]
    out_ref[...] = (1.0 - PRESERVE) * o + PRESERVE * xp_ref[...]


_fin = pl.pallas_call(
    _fin_body,
    grid=(NB,),
    in_specs=[
        pl.BlockSpec((1, BN, DH), lambda i: (0, i, 0)),
        pl.BlockSpec((1, BN, DH), lambda i: (1, i, 0)),
        pl.BlockSpec((BN, 1), lambda i: (i, 0)),
        pl.BlockSpec((1, D), lambda i: (0, 0)),
        pl.BlockSpec((BN, D), lambda i: (i, 0)),
    ],
    out_specs=pl.BlockSpec((BN, D), lambda i: (i, 0)),
    out_shape=jax.ShapeDtypeStruct((N, D), jnp.float32),
)


def kernel(skill_embed, adj_list, edge_attr, W1, b1, W2, b2):
    pad = EP - EA
    loop = jnp.arange(N, dtype=jnp.int32)
    src = jnp.concatenate([adj_list[0], loop, jnp.zeros((pad,), jnp.int32)])
    dst = jnp.concatenate([adj_list[1], loop, jnp.zeros((pad,), jnp.int32)])
    w = jnp.concatenate([edge_attr, jnp.ones((N,), jnp.float32),
                         jnp.zeros((pad,), jnp.float32)])
    srcp = src.reshape(ER, 128)
    dstp = dst.reshape(ER, 128)
    wp = w.reshape(ER, 128)
    degp = _deg_kernel(dstp, wp)
    degc = degp[0].reshape(NP)[:N].reshape(N, 1)

    h1 = _mm(skill_embed, W1, degc)
    s1 = _edge_kernel(h1.reshape(NC * NP, DH), srcp, dstp, wp)
    t1 = _fin(s1, s1, degc, b1.reshape(1, D), skill_embed)

    h2 = _mm(t1, W2, degc)
    s2 = _edge_kernel(h2.reshape(NC * NP, DH), srcp, dstp, wp)
    out = _fin(s2, s2, degc, b2.reshape(1, D), t1)
    return out


# speedup vs baseline: 1.8548x; 1.8548x over previous
import functools

import jax
import jax.numpy as jnp
from jax import lax
from jax.experimental import pallas as pl
from jax.experimental.pallas import tpu as pltpu
from jax.experimental.pallas import tpu_sc as plsc

N = 10000
D = 256
DH = 128
E = 160000
L = 16
NC = 2
NS = 16
EP = 163840
ER = EP // 128
ROWS_T = ER // NS
NP = 10240
SEG = NP // NS
NSUB = N // NS
BLK = 16
NB = 10
BN = N // NB
PRESERVE = 0.1

_mesh = plsc.VectorSubcoreMesh(
    core_axis_name="c", subcore_axis_name="s", num_cores=NC, num_subcores=NS)
_sc_params = pltpu.CompilerParams(needs_layout_passes=False)


@functools.partial(
    pl.kernel,
    out_type=jax.ShapeDtypeStruct((NC, NP // 128, 128), jnp.float32),
    mesh=_mesh,
    scratch_types=[
        pltpu.VMEM((NP // 128, 128), jnp.float32),
        pltpu.VMEM((ROWS_T, 128), jnp.int32),
        pltpu.VMEM((ROWS_T, 128), jnp.float32),
        pltpu.VMEM_SHARED((NS, NP // 128, 128), jnp.float32),
        pltpu.VMEM((NS, 8, 128), jnp.float32),
        pltpu.VMEM((8, 128), jnp.float32),
    ],
    compiler_params=_sc_params,
)
def _deg_kernel(dst_hbm, w_hbm, deg_hbm, degloc, dstv, wv, degsh, redv, accv):
    c = lax.axis_index("c")
    s = lax.axis_index("s")
    zero = jnp.zeros((L,), jnp.float32)

    def zbody(i, _):
        for q in range(128 // L):
            degloc[i, pl.ds(q * L, L)] = zero
        return 0
    lax.fori_loop(0, NP // 128, zbody, 0)

    pltpu.sync_copy(dst_hbm.at[pl.ds(s * ROWS_T, ROWS_T)], dstv)
    pltpu.sync_copy(w_hbm.at[pl.ds(s * ROWS_T, ROWS_T)], wv)

    def ebody(j, _):
        for q in range(128 // L):
            idx = dstv[j, pl.ds(q * L, L)]
            wq = wv[j, pl.ds(q * L, L)]
            plsc.addupdate_scatter(
                degloc, [lax.shift_right_logical(idx, 7),
                         lax.bitwise_and(idx, 127)], wq)
        return 0
    lax.fori_loop(0, ROWS_T, ebody, 0)

    pltpu.sync_copy(degloc, degsh.at[s])
    plsc.subcore_barrier()
    nrt = 8

    @pl.when(s < NP // 128 // nrt)
    def _reduce():
        pltpu.sync_copy(degsh.at[:, pl.ds(s * nrt, nrt)], redv)

        def rbody(v, _):
            for q in range(128 // L):
                acc = redv[0, v, pl.ds(q * L, L)]
                for r in range(1, NS):
                    acc = acc + redv[r, v, pl.ds(q * L, L)]
                accv[v, pl.ds(q * L, L)] = acc
            return 0
        lax.fori_loop(0, nrt, rbody, 0)
        pltpu.sync_copy(accv, deg_hbm.at[c, pl.ds(s * nrt, nrt)])


@functools.partial(
    pl.kernel,
    out_type=jax.ShapeDtypeStruct((NC, NP, DH), jnp.float32),
    mesh=_mesh,
    scratch_types=[
        pltpu.VMEM((ROWS_T, 128), jnp.int32),
        pltpu.VMEM((BLK, 128), jnp.int32),
        pltpu.VMEM((BLK, 128), jnp.float32),
        pltpu.VMEM((128, DH), jnp.float32),
        pltpu.VMEM((128, DH), jnp.float32),
        pltpu.VMEM_SHARED((NP, DH), jnp.float32),
        pltpu.SemaphoreType.DMA,
        pltpu.SemaphoreType.DMA,
        pltpu.SemaphoreType.DMA,
        pltpu.SemaphoreType.DMA,
    ],
    compiler_params=_sc_params,
)
def _edge_kernel(h_hbm, src_hbm, dst_hbm, w_hbm, out_hbm,
                 srcv, dblk, wblk, rows0, rows1, acc,
                 gsem0, gsem1, ssem0, ssem1):
    c = lax.axis_index("c")
    s = lax.axis_index("s")
    zero = jnp.zeros((L,), jnp.float32)
    nacc = NP // NS
    rows = (rows0, rows1)
    gsem = (gsem0, gsem1)
    ssem = (ssem0, ssem1)

    def zb(i, _):
        for q in range(DH // L):
            rows0[i, pl.ds(q * L, L)] = zero
        return 0
    lax.fori_loop(0, 128, zb, 0)
    for k in range(nacc // 128):
        pltpu.sync_copy(rows0, acc.at[pl.ds(s * nacc + k * 128, 128)])

    pltpu.sync_copy(src_hbm.at[pl.ds(s * ROWS_T, ROWS_T)], srcv)

    coff = c * N

    def ob(j, _):
        for q in range(128 // L):
            srcv[j, pl.ds(q * L, L)] = srcv[j, pl.ds(q * L, L)] + coff
        return 0
    lax.fori_loop(0, ROWS_T, ob, 0)

    plsc.subcore_barrier()

    def _wait(sem, buf):
        pltpu.make_async_copy(h_hbm.at[pl.ds(0, 128)], buf, sem).wait()

    for hh in range(2):
        pltpu.async_copy(h_hbm.at[srcv.at[0, pl.ds(hh * 64, 64)]],
                         rows0.at[pl.ds(hh * 64, 64)], gsem0)

    def pair(g, _):
        for b01 in range(2):
            j = 2 * g + b01
            bb = rows[b01]
            ob_ = rows[1 - b01]

            if b01 == 1:
                _wait(ssem[0], ob_)
            else:
                @pl.when(g >= 1)
                def _():
                    _wait(ssem[1], ob_)

            @pl.when(lax.rem(j, BLK) == 0)
            def _():
                blk = lax.div(j, BLK)
                pltpu.sync_copy(
                    dst_hbm.at[pl.ds(s * ROWS_T + blk * BLK, BLK)], dblk)
                pltpu.sync_copy(
                    w_hbm.at[pl.ds(s * ROWS_T + blk * BLK, BLK)], wblk)

            def _fire_gather(jn, dst_slot, sem):
                for hh in range(2):
                    pltpu.async_copy(
                        h_hbm.at[srcv.at[jn, pl.ds(hh * 64, 64)]],
                        dst_slot.at[pl.ds(hh * 64, 64)], sem)

            if b01 == 0:
                _fire_gather(j + 1, ob_, gsem[1])
            else:
                @pl.when(g < ROWS_T // 2 - 1)
                def _():
                    _fire_gather(j + 1, ob_, gsem[0])

            _wait(gsem[b01], bb)
            jm = lax.rem(j, BLK)

            def scale(r4, _2):
                jj = jnp.broadcast_to(jm, (L,))
                wspl = []
                for u in range(4):
                    r = r4 * 4 + u
                    rr = jnp.broadcast_to(r, (L,))
                    wspl.append(plsc.load_gather(wblk, [jj, rr]))
                for u in range(4):
                    r = r4 * 4 + u
                    for q in range(DH // L):
                        bb[r, pl.ds(q * L, L)] = bb[r, pl.ds(q * L, L)] * wspl[u]
                return 0
            lax.fori_loop(0, 32, scale, 0)
            pltpu.async_copy(bb, acc.at[dblk.at[jm]], ssem[b01], add=True)
        return 0
    lax.fori_loop(0, ROWS_T // 2, pair, 0)
    _wait(ssem[1], rows1)

    plsc.subcore_barrier()
    pltpu.sync_copy(acc.at[pl.ds(s * nacc, nacc)],
                    out_hbm.at[c, pl.ds(s * nacc, nacc)])


def _mm_body(x_ref, w_ref, deg_ref, out_ref):
    dinv = lax.rsqrt(deg_ref[...] + 1.0)
    h = jnp.dot(x_ref[...], w_ref[...], preferred_element_type=jnp.float32)
    out_ref[...] = h * dinv


_mm = pl.pallas_call(
    _mm_body,
    grid=(NC, NB),
    in_specs=[
        pl.BlockSpec((BN, D), lambda c, i: (i, 0)),
        pl.BlockSpec((D, DH), lambda c, i: (0, c)),
        pl.BlockSpec((BN, 1), lambda c, i: (i, 0)),
    ],
    out_specs=pl.BlockSpec((BN, DH), lambda c, i: (c * NB + i, 0)),
    out_shape=jax.ShapeDtypeStruct((NC * N, DH), jnp.float32),
)


def _fin_body(s_lo, s_hi, h_lo, h_hi, deg_ref, b_ref, xp_ref, out_ref):
    dinv = lax.rsqrt(deg_ref[...] + 1.0)
    lo = dinv * (s_lo[0] + h_lo[...])
    hi = dinv * (s_hi[0] + h_hi[...])
    o = jnp.concatenate([lo, hi], axis=1) + b_ref[...]
    out_ref[...] = (1.0 - PRESERVE) * o + PRESERVE * xp_ref[...]


_fin = pl.pallas_call(
    _fin_body,
    grid=(NB,),
    in_specs=[
        pl.BlockSpec((1, BN, DH), lambda i: (0, i, 0)),
        pl.BlockSpec((1, BN, DH), lambda i: (1, i, 0)),
        pl.BlockSpec((BN, DH), lambda i: (i, 0)),
        pl.BlockSpec((BN, DH), lambda i: (NB + i, 0)),
        pl.BlockSpec((BN, 1), lambda i: (i, 0)),
        pl.BlockSpec((1, D), lambda i: (0, 0)),
        pl.BlockSpec((BN, D), lambda i: (i, 0)),
    ],
    out_specs=pl.BlockSpec((BN, D), lambda i: (i, 0)),
    out_shape=jax.ShapeDtypeStruct((N, D), jnp.float32),
)


def _fm_body(s_lo, s_hi, h_lo, h_hi, deg_ref, b_ref, xp_ref, w_ref,
             t_ref, out_ref):
    dinv = lax.rsqrt(deg_ref[...] + 1.0)
    lo = dinv * (s_lo[0] + h_lo[...])
    hi = dinv * (s_hi[0] + h_hi[...])
    o = jnp.concatenate([lo, hi], axis=1) + b_ref[...]
    t = (1.0 - PRESERVE) * o + PRESERVE * xp_ref[...]
    t_ref[...] = t
    h = jnp.dot(t, w_ref[...], preferred_element_type=jnp.float32)
    out_ref[...] = h * dinv


_fm = pl.pallas_call(
    _fm_body,
    grid=(NC, NB),
    in_specs=[
        pl.BlockSpec((1, BN, DH), lambda c, i: (0, i, 0)),
        pl.BlockSpec((1, BN, DH), lambda c, i: (1, i, 0)),
        pl.BlockSpec((BN, DH), lambda c, i: (i, 0)),
        pl.BlockSpec((BN, DH), lambda c, i: (NB + i, 0)),
        pl.BlockSpec((BN, 1), lambda c, i: (i, 0)),
        pl.BlockSpec((1, D), lambda c, i: (0, 0)),
        pl.BlockSpec((BN, D), lambda c, i: (i, 0)),
        pl.BlockSpec((D, DH), lambda c, i: (0, c)),
    ],
    out_specs=[
        pl.BlockSpec((BN, D), lambda c, i: (i, 0)),
        pl.BlockSpec((BN, DH), lambda c, i: (c * NB + i, 0)),
    ],
    out_shape=[
        jax.ShapeDtypeStruct((N, D), jnp.float32),
        jax.ShapeDtypeStruct((NC * N, DH), jnp.float32),
    ],
)


def kernel(skill_embed, adj_list, edge_attr, W1, b1, W2, b2):
    pad = EP - E
    src = jnp.concatenate([adj_list[0], jnp.zeros((pad,), jnp.int32)])
    dst = jnp.concatenate([adj_list[1], jnp.zeros((pad,), jnp.int32)])
    w = jnp.concatenate([edge_attr, jnp.zeros((pad,), jnp.float32)])
    srcp = src.reshape(ER, 128)
    dstp = dst.reshape(ER, 128)
    wp = w.reshape(ER, 128)

    degp = _deg_kernel(dstp, wp)
    degc = degp[0].reshape(NP)[:N].reshape(N, 1)

    h1 = _mm(skill_embed, W1, degc)
    s1 = _edge_kernel(h1, srcp, dstp, wp)
    t1, h2 = _fm(s1, s1, h1, h1, degc, b1.reshape(1, D), skill_embed, W2)
    s2 = _edge_kernel(h2, srcp, dstp, wp)
    out = _fin(s2, s2, h2, h2, degc, b2.reshape(1, D), t1)
    return out
